# baseline (speedup 1.0000x reference)
import jax
import jax.numpy as jnp
from jax.experimental import pallas as pl
from jax.experimental.pallas import tpu as pltpu

_CHUNK = 512
_SPLIT = 2
_RING = 8
_WM_CHUNKS = 8


def _router_kernel(x_hbm, wm_hbm, bm_ref, wr_ref, br_ref, out_ref,
                   wm_vmem, xbuf, wf_ref, wm_sem, x_sem):
    d_model = x_hbm.shape[1]
    n_chunks = x_hbm.shape[0] // _CHUNK
    wm_rows = d_model // _WM_CHUNKS
    sub = _CHUNK // _SPLIT

    def wm_copy(k):
        return pltpu.make_async_copy(
            wm_hbm.at[pl.ds(k * wm_rows, wm_rows), :],
            wm_vmem.at[pl.ds(k * wm_rows, wm_rows), :],
            wm_sem)

    def x_copies(j):
        s = j % _RING
        return [pltpu.make_async_copy(
                    x_hbm.at[pl.ds(j * _CHUNK + p * sub, sub), :],
                    xbuf.at[s, pl.ds(p * sub, sub), :],
                    x_sem.at[s])
                for p in range(_SPLIT)]

    def x_start(j):
        for c in x_copies(j):
            c.start()

    def x_wait(j):
        for c in x_copies(j):
            c.wait()

    for k in range(_WM_CHUNKS):
        wm_copy(k).start()
    for j in range(min(_RING, n_chunks)):
        x_start(j)

    for k in range(_WM_CHUNKS):
        wm_copy(k).wait()
    wf_ref[...] = jnp.dot(wm_vmem[...], wr_ref[...],
                          preferred_element_type=jnp.float32)
    bf = jnp.dot(bm_ref[...], wr_ref[...],
                 preferred_element_type=jnp.float32) + br_ref[...]

    for j in range(n_chunks):
        x_wait(j)
        logits = jnp.dot(xbuf[j % _RING], wf_ref[...],
                         preferred_element_type=jnp.float32) + bf
        m = jnp.max(logits, axis=-1, keepdims=True)
        e = jnp.exp(logits - m)
        out_ref[pl.ds(j * _CHUNK, _CHUNK), :] = (
            e / jnp.sum(e, axis=-1, keepdims=True))
        if j + _RING < n_chunks:
            x_start(j + _RING)


def kernel(x, W_model, b_model, W_router, b_router):
    tokens, d_model = x.shape
    h_out = W_model.shape[1]
    n_experts = W_router.shape[1]
    bm = b_model.reshape(1, h_out)
    br = b_router.reshape(1, n_experts)
    return pl.pallas_call(
        _router_kernel,
        in_specs=[
            pl.BlockSpec(memory_space=pl.ANY),
            pl.BlockSpec(memory_space=pl.ANY),
            pl.BlockSpec((1, h_out), lambda: (0, 0)),
            pl.BlockSpec((h_out, n_experts), lambda: (0, 0)),
            pl.BlockSpec((1, n_experts), lambda: (0, 0)),
        ],
        out_specs=pl.BlockSpec((tokens, n_experts), lambda: (0, 0)),
        out_shape=jax.ShapeDtypeStruct((tokens, n_experts), jnp.float32),
        scratch_shapes=[
            pltpu.VMEM((d_model, h_out), jnp.float32),
            pltpu.VMEM((_RING, _CHUNK, d_model), jnp.float32),
            pltpu.VMEM((d_model, n_experts), jnp.float32),
            pltpu.SemaphoreType.DMA,
            pltpu.SemaphoreType.DMA((_RING,)),
        ],
    )(x, W_model, bm, W_router, br)


# speedup vs baseline: 1.3570x; 1.3570x over previous
import jax
import jax.numpy as jnp
from jax.experimental import pallas as pl
from jax.experimental.pallas import tpu as pltpu

_TOKEN_BLOCK = 2048


def _router_kernel(x_ref, wm_ref, bm_ref, wr_ref, br_ref, out_ref,
                   wf_ref, bf_ref):
    @pl.when(pl.program_id(0) == 0)
    def _fuse_weights():
        wf_ref[...] = jnp.dot(wm_ref[...], wr_ref[...],
                              preferred_element_type=jnp.float32)
        bf_ref[...] = jnp.dot(bm_ref[...], wr_ref[...],
                              preferred_element_type=jnp.float32) + br_ref[...]

    logits = jnp.dot(x_ref[...], wf_ref[...],
                     preferred_element_type=jnp.float32)
    logits = logits + bf_ref[...]
    m = jnp.max(logits, axis=-1, keepdims=True)
    e = jnp.exp(logits - m)
    out_ref[...] = e / jnp.sum(e, axis=-1, keepdims=True)


def kernel(x, W_model, b_model, W_router, b_router):
    tokens, d_model = x.shape
    h_out = W_model.shape[1]
    n_experts = W_router.shape[1]
    tb = min(_TOKEN_BLOCK, tokens)
    bm = b_model.reshape(1, h_out)
    br = b_router.reshape(1, n_experts)
    return pl.pallas_call(
        _router_kernel,
        grid=(tokens // tb,),
        in_specs=[
            pl.BlockSpec((tb, d_model), lambda i: (i, 0)),
            pl.BlockSpec((d_model, h_out), lambda i: (0, 0)),
            pl.BlockSpec((1, h_out), lambda i: (0, 0)),
            pl.BlockSpec((h_out, n_experts), lambda i: (0, 0)),
            pl.BlockSpec((1, n_experts), lambda i: (0, 0)),
        ],
        out_specs=pl.BlockSpec((tb, n_experts), lambda i: (i, 0)),
        out_shape=jax.ShapeDtypeStruct((tokens, n_experts), jnp.float32),
        scratch_shapes=[
            pltpu.VMEM((d_model, n_experts), jnp.float32),
            pltpu.VMEM((1, n_experts), jnp.float32),
        ],
    )(x, W_model, bm, W_router, br)
